# baseline (speedup 1.0000x reference)
import jax
import jax.numpy as jnp
from jax import lax
from jax.experimental import pallas as pl
from jax.experimental.pallas import tpu as pltpu

_NCENT = 512
_NCB = 32
_SUB = 64
_BT = 128
_DIN = _NCB * _SUB
_AUG = _SUB + 2


def _fused_body(x_ref, caug_ref, w_ref, prod_ref, negmse_ref,
                codes_ref, outs_ref):
    x = x_ref[...]
    negs = []
    codes = []
    for c in range(_NCB):
        xc = x[:, c * _SUB:(c + 1) * _SUB]
        cc = caug_ref[c][:, :_SUB]
        resp = lax.dot_general(xc, cc, (((1,), (1,)), ((), ())))
        m = jnp.max(resp, axis=-1, keepdims=True)
        eqf = (resp >= m).astype(jnp.float32)
        g = lax.dot_general(eqf, caug_ref[c], (((1,), (0,)), ((), ())),
                            preferred_element_type=jnp.float32)
        outs_ref[:, c * _SUB:(c + 1) * _SUB] = g[:, :_SUB]
        code = (g[:, _SUB:_SUB + 1] * 16.0 + g[:, _SUB + 1:_SUB + 2])
        codes.append(code.astype(jnp.int32))
        negs.append(-m)
    negmse_ref[...] = jnp.concatenate(negs, axis=1)
    codes_ref[...] = jnp.concatenate(codes, axis=1)
    prod_ref[...] = lax.dot_general(outs_ref[...], w_ref[...],
                                    (((1,), (0,)), ((), ())),
                                    preferred_element_type=jnp.float32)


def kernel(inputs, centroids, W):
    B = inputs.shape[0]
    out_dim = W.shape[1]
    x2 = inputs.reshape(B, _DIN)
    k = jnp.arange(_NCENT, dtype=jnp.int32)
    khi = (k // 16).astype(jnp.float32)
    klo = (k % 16).astype(jnp.float32)
    idx_cols = jnp.stack([khi, klo], axis=1)
    caug = jnp.concatenate(
        [centroids,
         jnp.broadcast_to(idx_cols[None], (_NCB, _NCENT, 2))], axis=2)
    grid = (B // _BT,)
    prod, negmse, codes = pl.pallas_call(
        _fused_body,
        grid=grid,
        in_specs=[
            pl.BlockSpec((_BT, _DIN), lambda i: (i, 0)),
            pl.BlockSpec((_NCB, _NCENT, _AUG), lambda i: (0, 0, 0)),
            pl.BlockSpec((_DIN, out_dim), lambda i: (0, 0)),
        ],
        out_specs=(
            pl.BlockSpec((_BT, out_dim), lambda i: (i, 0)),
            pl.BlockSpec((_BT, _NCB), lambda i: (i, 0)),
            pl.BlockSpec((_BT, _NCB), lambda i: (i, 0)),
        ),
        out_shape=(
            jax.ShapeDtypeStruct((B, out_dim), jnp.float32),
            jax.ShapeDtypeStruct((B, _NCB), jnp.float32),
            jax.ShapeDtypeStruct((B, _NCB), jnp.int32),
        ),
        scratch_shapes=[pltpu.VMEM((_BT, _DIN), jnp.float32)],
        compiler_params=pltpu.CompilerParams(
            dimension_semantics=("arbitrary",),
        ),
    )(x2, caug, W)
    return (prod, negmse, codes)


# speedup vs baseline: 2.0387x; 2.0387x over previous
import jax
import jax.numpy as jnp
from jax import lax
from jax.experimental import pallas as pl
from jax.experimental.pallas import tpu as pltpu

_NCENT = 512
_NCB = 32
_SUB = 64
_BT = 256
_DIN = _NCB * _SUB
_AUG = _SUB + 2


def _fused_body(x_ref, caug_ref, w_ref, prod_ref, negmse_ref,
                codes_ref, outs_ref):
    x = x_ref[...]
    negs = []
    codes = []
    for c in range(_NCB):
        xc = x[:, c * _SUB:(c + 1) * _SUB]
        cc = caug_ref[c][:, :_SUB]
        resp = lax.dot_general(xc, cc, (((1,), (1,)), ((), ())))
        m = jnp.max(resp, axis=-1, keepdims=True)
        eqf = (resp >= m).astype(jnp.float32)
        g = lax.dot_general(eqf, caug_ref[c], (((1,), (0,)), ((), ())),
                            preferred_element_type=jnp.float32)
        outs_ref[:, c * _SUB:(c + 1) * _SUB] = g[:, :_SUB]
        code = (g[:, _SUB:_SUB + 1] * 16.0 + g[:, _SUB + 1:_SUB + 2])
        codes.append(code.astype(jnp.int32))
        negs.append(-m)
    negmse_ref[...] = jnp.concatenate(negs, axis=1)
    codes_ref[...] = jnp.concatenate(codes, axis=1)
    prod_ref[...] = lax.dot_general(outs_ref[...], w_ref[...],
                                    (((1,), (0,)), ((), ())),
                                    preferred_element_type=jnp.float32)


def kernel(inputs, centroids, W):
    B = inputs.shape[0]
    out_dim = W.shape[1]
    x2 = inputs.reshape(B, _DIN)
    k = jnp.arange(_NCENT, dtype=jnp.int32)
    khi = (k // 16).astype(jnp.float32)
    klo = (k % 16).astype(jnp.float32)
    idx_cols = jnp.stack([khi, klo], axis=1)
    caug = jnp.concatenate(
        [centroids,
         jnp.broadcast_to(idx_cols[None], (_NCB, _NCENT, 2))], axis=2)
    grid = (B // _BT,)
    prod, negmse, codes = pl.pallas_call(
        _fused_body,
        grid=grid,
        in_specs=[
            pl.BlockSpec((_BT, _DIN), lambda i: (i, 0)),
            pl.BlockSpec((_NCB, _NCENT, _AUG), lambda i: (0, 0, 0)),
            pl.BlockSpec((_DIN, out_dim), lambda i: (0, 0)),
        ],
        out_specs=(
            pl.BlockSpec((_BT, out_dim), lambda i: (i, 0)),
            pl.BlockSpec((_BT, _NCB), lambda i: (i, 0)),
            pl.BlockSpec((_BT, _NCB), lambda i: (i, 0)),
        ),
        out_shape=(
            jax.ShapeDtypeStruct((B, out_dim), jnp.float32),
            jax.ShapeDtypeStruct((B, _NCB), jnp.float32),
            jax.ShapeDtypeStruct((B, _NCB), jnp.int32),
        ),
        scratch_shapes=[pltpu.VMEM((_BT, _DIN), jnp.float32)],
        compiler_params=pltpu.CompilerParams(
            dimension_semantics=("arbitrary",),
        ),
    )(x2, caug, W)
    return (prod, negmse, codes)
